# baseline (speedup 1.0000x reference)
import functools

import jax
import jax.numpy as jnp
from jax import lax
from jax.experimental import pallas as pl
from jax.experimental.pallas import tpu as pltpu
from jax.experimental.pallas import tpu_sc as plsc

H, W, C = 1080, 1920, 3
HW = H * W
BLK = 8
NTILES = 32
SIZE = HW // NTILES
WIN = 12800
NWIN = HW // WIN
CH = 6480
NPAD = 1024


def _tc_prep(fx_ref, fy_ref, depth_ref, tlin_ref, maxd_ref):
    i = pl.program_id(0)
    fx = fx_ref[...]
    fy = fy_ref[...]
    xs = lax.broadcasted_iota(jnp.int32, (BLK, W), 1).astype(jnp.float32)
    ys = (lax.broadcasted_iota(jnp.int32, (BLK, W), 0) + i * BLK).astype(
        jnp.float32)
    tx = jnp.round(xs + fx).astype(jnp.int32)
    ty = jnp.round(ys + fy).astype(jnp.int32)
    valid = (tx >= 0) & (tx < W) & (ty >= 0) & (ty < H)
    tlin_ref[...] = jnp.where(valid, ty * W + tx, HW)
    bm = jnp.max(depth_ref[...])

    @pl.when(i == 0)
    def _():
        maxd_ref[0, 0] = bm

    @pl.when(i > 0)
    def _():
        maxd_ref[0, 0] = jnp.maximum(maxd_ref[0, 0], bm)


def _sc_scatter_body(tlin_hbm, winner_hbm, wloc, buf):
    wid = lax.axis_index("s") * 2 + lax.axis_index("c")
    base = wid * SIZE
    neg1 = jnp.full((16,), -1, jnp.int32)

    def init_body(j, _):
        wloc[pl.ds(j * 16, 16)] = neg1
        return ()

    lax.fori_loop(0, SIZE // 16, init_body, ())
    iota16 = lax.iota(jnp.int32, 16)

    def win_body(w, _):
        pltpu.sync_copy(tlin_hbm.at[pl.ds(w * WIN, WIN)], buf)

        def vbody(j, _):
            t = buf[pl.ds(j * 16, 16)]
            local = t - base
            inr = (local >= 0) & (local < SIZE)
            _, lastocc = plsc.scan_count(local, inr)
            keep = lastocc & inr
            sval = (w * WIN + j * 16) + iota16
            plsc.store_scatter(wloc, [local], sval, mask=keep)
            return ()

        lax.fori_loop(0, WIN // 16, vbody, ())
        return ()

    lax.fori_loop(0, NWIN, win_body, ())
    pltpu.sync_copy(wloc, winner_hbm.at[pl.ds(base, SIZE)])


def _sc_gather_body(winner_hbm, depth_hbm, imgz_hbm, maxd_hbm, out_hbm,
                    wbuf, dbuf, idx0, idx1, idx2, r0, r1, r2, rows, maxd_v,
                    sem):
    wid = lax.axis_index("s") * 2 + lax.axis_index("c")
    base = wid * SIZE
    pltpu.sync_copy(maxd_hbm, maxd_v)
    iota16 = lax.iota(jnp.int32, 16)

    def chunk_body(k, _):
        off = base + k * CH
        pltpu.sync_copy(winner_hbm.at[pl.ds(off, CH)], wbuf)

        def clip_body(j, _):
            w = wbuf[pl.ds(j * 16, 16)]
            idx0[pl.ds(j * 16, 16)] = jnp.maximum(w, 0)
            return ()

        lax.fori_loop(0, CH // 16, clip_body, ())
        pltpu.async_copy(depth_hbm.at[idx0], dbuf, sem).wait()
        mv = maxd_v[...]

        def fix_body(j, _):
            w = wbuf[pl.ds(j * 16, 16)]
            d = dbuf[pl.ds(j * 16, 16)]
            good = (w >= 0) & (d != mv)
            pad = HW + ((j * 16 + iota16) & (NPAD - 1))
            fidx3 = jnp.where(good, w, pad) * 3
            idx0[pl.ds(j * 16, 16)] = fidx3
            idx1[pl.ds(j * 16, 16)] = fidx3 + 1
            idx2[pl.ds(j * 16, 16)] = fidx3 + 2
            return ()

        lax.fori_loop(0, CH // 16, fix_body, ())
        pltpu.async_copy(imgz_hbm.at[idx0], r0, sem).wait()
        pltpu.async_copy(imgz_hbm.at[idx1], r1, sem).wait()
        pltpu.async_copy(imgz_hbm.at[idx2], r2, sem).wait()

        def mix_body(j, _):
            pos = (j * 16 + iota16) * 3
            plsc.store_scatter(rows, [pos], r0[pl.ds(j * 16, 16)])
            plsc.store_scatter(rows, [pos + 1], r1[pl.ds(j * 16, 16)])
            plsc.store_scatter(rows, [pos + 2], r2[pl.ds(j * 16, 16)])
            return ()

        lax.fori_loop(0, CH // 16, mix_body, ())
        pltpu.sync_copy(rows, out_hbm.at[pl.ds(off * 3, CH * 3)])
        return ()

    lax.fori_loop(0, SIZE // CH, chunk_body, ())


@jax.jit
def kernel(img, flow, depth, split):
    fx = flow[0, :, :, 0]
    fy = flow[0, :, :, 1]
    tlin, maxd = pl.pallas_call(
        _tc_prep,
        grid=(H // BLK,),
        in_specs=[
            pl.BlockSpec((BLK, W), lambda i: (i, 0)),
            pl.BlockSpec((BLK, W), lambda i: (i, 0)),
            pl.BlockSpec((BLK, W), lambda i: (i, 0)),
        ],
        out_specs=[
            pl.BlockSpec((BLK, W), lambda i: (i, 0)),
            pl.BlockSpec(memory_space=pltpu.SMEM, block_shape=(1, 1),
                         index_map=lambda i: (0, 0)),
        ],
        out_shape=[
            jax.ShapeDtypeStruct((H, W), jnp.int32),
            jax.ShapeDtypeStruct((1, 1), jnp.float32),
        ],
    )(fx, fy, depth)

    mesh = plsc.VectorSubcoreMesh(core_axis_name="c", subcore_axis_name="s")

    sc_scatter = pl.kernel(
        _sc_scatter_body,
        out_type=jax.ShapeDtypeStruct((HW,), jnp.int32),
        mesh=mesh,
        scratch_types=[
            pltpu.VMEM((SIZE,), jnp.int32),
            pltpu.VMEM((WIN,), jnp.int32),
        ],
        compiler_params=pltpu.CompilerParams(needs_layout_passes=False),
    )
    winner = sc_scatter(tlin.reshape(HW))

    imgz = jnp.concatenate(
        [img.reshape(HW, C), jnp.zeros((NPAD, C), jnp.float32)],
        axis=0).reshape((HW + NPAD) * C)
    maxd16 = jnp.broadcast_to(maxd.reshape(()), (16,))

    sc_gather = pl.kernel(
        _sc_gather_body,
        out_type=jax.ShapeDtypeStruct((HW * C,), jnp.float32),
        mesh=mesh,
        scratch_types=[
            pltpu.VMEM((CH,), jnp.int32),
            pltpu.VMEM((CH,), jnp.float32),
            pltpu.VMEM((CH,), jnp.int32),
            pltpu.VMEM((CH,), jnp.int32),
            pltpu.VMEM((CH,), jnp.int32),
            pltpu.VMEM((CH,), jnp.float32),
            pltpu.VMEM((CH,), jnp.float32),
            pltpu.VMEM((CH,), jnp.float32),
            pltpu.VMEM((CH * C,), jnp.float32),
            pltpu.VMEM((16,), jnp.float32),
            pltpu.SemaphoreType.DMA,
        ],
        compiler_params=pltpu.CompilerParams(needs_layout_passes=False),
    )
    out = sc_gather(winner, depth.reshape(HW), imgz, maxd16)
    return out.reshape(H, W, C)


# speedup vs baseline: 1.3558x; 1.3558x over previous
import jax
import jax.numpy as jnp
from jax import lax
from jax.experimental import pallas as pl
from jax.experimental.pallas import tpu as pltpu
from jax.experimental.pallas import tpu_sc as plsc

H, W, C = 1080, 1920, 3
HW = H * W
BLK = 8
NTILES = 32
SIZE = HW // NTILES
WIN = 12800
NWIN = HW // WIN
CH = 6480
TMASK = (1 << 21) - 1
FLAG = 1 << 21


def _tc_maxd(depth_ref, maxd_ref):
    i = pl.program_id(0)
    bm = jnp.max(depth_ref[...])

    @pl.when(i == 0)
    def _():
        maxd_ref[0, 0] = bm

    @pl.when(i > 0)
    def _():
        maxd_ref[0, 0] = jnp.maximum(maxd_ref[0, 0], bm)


def _tc_prep(maxd_ref, fx_ref, fy_ref, depth_ref, tlin_ref):
    i = pl.program_id(0)
    fx = fx_ref[...]
    fy = fy_ref[...]
    xs = lax.broadcasted_iota(jnp.int32, (BLK, W), 1).astype(jnp.float32)
    ys = (lax.broadcasted_iota(jnp.int32, (BLK, W), 0) + i * BLK).astype(
        jnp.float32)
    tx = jnp.round(xs + fx).astype(jnp.int32)
    ty = jnp.round(ys + fy).astype(jnp.int32)
    valid = (tx >= 0) & (tx < W) & (ty >= 0) & (ty < H)
    bad = (depth_ref[...] == maxd_ref[0, 0]).astype(jnp.int32)
    tlin_ref[...] = jnp.where(valid, ty * W + tx, HW) | (bad << 21)


def _sc_scatter_body(tlin_hbm, winner_hbm, wloc, buf):
    wid = lax.axis_index("s") * 2 + lax.axis_index("c")
    base = wid * SIZE
    neg1 = jnp.full((16,), -1, jnp.int32)

    def init_body(j, _):
        wloc[pl.ds(j * 16, 16)] = neg1
        return ()

    lax.fori_loop(0, SIZE // 16, init_body, ())
    iota16 = lax.iota(jnp.int32, 16)

    def win_body(w, _):
        pltpu.sync_copy(tlin_hbm.at[pl.ds(w * WIN, WIN)], buf)

        def vbody(j, _):
            t = buf[pl.ds(j * 16, 16)]
            local = (t & TMASK) - base
            inr = (local >= 0) & (local < SIZE)
            _, lastocc = plsc.scan_count(local, inr)
            keep = lastocc & inr
            s = (w * WIN + j * 16) + iota16
            sval = s * 2 + (t >> 21)
            plsc.store_scatter(wloc, [local], sval, mask=keep)
            return ()

        lax.fori_loop(0, WIN // 16, vbody, ())
        return ()

    lax.fori_loop(0, NWIN, win_body, ())
    pltpu.sync_copy(wloc, winner_hbm.at[pl.ds(base, SIZE)])


def _sc_gather_body(winner_hbm, img_hbm, out_hbm,
                    wbuf, idx0, idx1, idx2, r0, r1, r2, rows, sem):
    wid = lax.axis_index("s") * 2 + lax.axis_index("c")
    base = wid * SIZE
    iota16 = lax.iota(jnp.int32, 16)

    def chunk_body(k, _):
        off = base + k * CH
        pltpu.sync_copy(winner_hbm.at[pl.ds(off, CH)], wbuf)

        def fix_body(j, _):
            w = wbuf[pl.ds(j * 16, 16)]
            good = (w & 1) == 0
            pos = off + j * 16 + iota16
            fidx3 = jnp.where(good, (w >> 1) * 3, pos * 3)
            idx0[pl.ds(j * 16, 16)] = fidx3
            idx1[pl.ds(j * 16, 16)] = fidx3 + 1
            idx2[pl.ds(j * 16, 16)] = fidx3 + 2
            return ()

        lax.fori_loop(0, CH // 16, fix_body, ())
        pltpu.async_copy(img_hbm.at[idx0], r0, sem).wait()
        pltpu.async_copy(img_hbm.at[idx1], r1, sem).wait()
        pltpu.async_copy(img_hbm.at[idx2], r2, sem).wait()
        zero16 = jnp.zeros((16,), jnp.float32)

        def mix_body(j, _):
            w = wbuf[pl.ds(j * 16, 16)]
            good = (w & 1) == 0
            pos3 = (j * 16 + iota16) * 3
            v0 = jnp.where(good, r0[pl.ds(j * 16, 16)], zero16)
            v1 = jnp.where(good, r1[pl.ds(j * 16, 16)], zero16)
            v2 = jnp.where(good, r2[pl.ds(j * 16, 16)], zero16)
            plsc.store_scatter(rows, [pos3], v0)
            plsc.store_scatter(rows, [pos3 + 1], v1)
            plsc.store_scatter(rows, [pos3 + 2], v2)
            return ()

        lax.fori_loop(0, CH // 16, mix_body, ())
        pltpu.sync_copy(rows, out_hbm.at[pl.ds(off * 3, CH * 3)])
        return ()

    lax.fori_loop(0, SIZE // CH, chunk_body, ())


@jax.jit
def kernel(img, flow, depth, split):
    fx = flow[0, :, :, 0]
    fy = flow[0, :, :, 1]
    maxd = pl.pallas_call(
        _tc_maxd,
        grid=(H // BLK,),
        in_specs=[pl.BlockSpec((BLK, W), lambda i: (i, 0))],
        out_specs=pl.BlockSpec(memory_space=pltpu.SMEM, block_shape=(1, 1),
                               index_map=lambda i: (0, 0)),
        out_shape=jax.ShapeDtypeStruct((1, 1), jnp.float32),
    )(depth)
    tlin = pl.pallas_call(
        _tc_prep,
        grid=(H // BLK,),
        in_specs=[
            pl.BlockSpec(memory_space=pltpu.SMEM, block_shape=(1, 1),
                         index_map=lambda i: (0, 0)),
            pl.BlockSpec((BLK, W), lambda i: (i, 0)),
            pl.BlockSpec((BLK, W), lambda i: (i, 0)),
            pl.BlockSpec((BLK, W), lambda i: (i, 0)),
        ],
        out_specs=pl.BlockSpec((BLK, W), lambda i: (i, 0)),
        out_shape=jax.ShapeDtypeStruct((H, W), jnp.int32),
    )(maxd, fx, fy, depth)

    mesh = plsc.VectorSubcoreMesh(core_axis_name="c", subcore_axis_name="s")

    sc_scatter = pl.kernel(
        _sc_scatter_body,
        out_type=jax.ShapeDtypeStruct((HW,), jnp.int32),
        mesh=mesh,
        scratch_types=[
            pltpu.VMEM((SIZE,), jnp.int32),
            pltpu.VMEM((WIN,), jnp.int32),
        ],
        compiler_params=pltpu.CompilerParams(needs_layout_passes=False),
    )
    winner = sc_scatter(tlin.reshape(HW))

    sc_gather = pl.kernel(
        _sc_gather_body,
        out_type=jax.ShapeDtypeStruct((HW * C,), jnp.float32),
        mesh=mesh,
        scratch_types=[
            pltpu.VMEM((CH,), jnp.int32),
            pltpu.VMEM((CH,), jnp.int32),
            pltpu.VMEM((CH,), jnp.int32),
            pltpu.VMEM((CH,), jnp.int32),
            pltpu.VMEM((CH,), jnp.float32),
            pltpu.VMEM((CH,), jnp.float32),
            pltpu.VMEM((CH,), jnp.float32),
            pltpu.VMEM((CH * C,), jnp.float32),
            pltpu.SemaphoreType.DMA,
        ],
        compiler_params=pltpu.CompilerParams(needs_layout_passes=False),
    )
    out = sc_gather(winner, img.reshape(HW * C))
    return out.reshape(H, W, C)


# speedup vs baseline: 3.2189x; 2.3741x over previous
import jax
import jax.numpy as jnp
from jax import lax
from jax.experimental import pallas as pl
from jax.experimental.pallas import tpu as pltpu
from jax.experimental.pallas import tpu_sc as plsc

H, W, C = 1080, 1920, 3
HW = H * W
BLK = 8
NTILES = 32
SIZE = HW // NTILES
WIN = 12800
NWIN = HW // WIN
CH = 6480
TMASK = (1 << 21) - 1
FLAG = 1 << 21


def _tc_maxd(depth_ref, maxd_ref):
    i = pl.program_id(0)
    bm = jnp.max(depth_ref[...])

    @pl.when(i == 0)
    def _():
        maxd_ref[0, 0] = bm

    @pl.when(i > 0)
    def _():
        maxd_ref[0, 0] = jnp.maximum(maxd_ref[0, 0], bm)


def _tc_prep(maxd_ref, fx_ref, fy_ref, depth_ref, tlin_ref):
    i = pl.program_id(0)
    fx = fx_ref[...]
    fy = fy_ref[...]
    xs = lax.broadcasted_iota(jnp.int32, (BLK, W), 1).astype(jnp.float32)
    ys = (lax.broadcasted_iota(jnp.int32, (BLK, W), 0) + i * BLK).astype(
        jnp.float32)
    tx = jnp.round(xs + fx).astype(jnp.int32)
    ty = jnp.round(ys + fy).astype(jnp.int32)
    valid = (tx >= 0) & (tx < W) & (ty >= 0) & (ty < H)
    bad = (depth_ref[...] == maxd_ref[0, 0]).astype(jnp.int32)
    tlin_ref[...] = jnp.where(valid, ty * W + tx, HW) | (bad << 21)


def _sc_scatter_body(tlin_hbm, winner_hbm, wloc, buf):
    wid = lax.axis_index("s") * 2 + lax.axis_index("c")
    base = wid * SIZE
    neg1 = jnp.full((16,), -1, jnp.int32)

    def init_body(j, _):
        wloc[pl.ds(j * 16, 16)] = neg1
        return ()

    lax.fori_loop(0, SIZE // 16, init_body, ())
    iota16 = lax.iota(jnp.int32, 16)

    def win_body(w, _):
        pltpu.sync_copy(tlin_hbm.at[pl.ds(w * WIN, WIN)], buf)

        def vbody(j, _):
            t = buf[pl.ds(j * 16, 16)]
            local = (t & TMASK) - base
            inr = (local >= 0) & (local < SIZE)
            _, lastocc = plsc.scan_count(local, inr)
            keep = lastocc & inr
            s = (w * WIN + j * 16) + iota16
            sval = s * 2 + (t >> 21)
            plsc.store_scatter(wloc, [local], sval, mask=keep)
            return ()

        lax.fori_loop(0, WIN // 16, vbody, ())
        return ()

    lax.fori_loop(0, NWIN, win_body, ())
    pltpu.sync_copy(wloc, winner_hbm.at[pl.ds(base, SIZE)])


def _sc_gather_body(winner_hbm, img0_hbm, img1_hbm, img2_hbm, out_hbm,
                    wbuf, idx0, r0, r1, r2, rows, sem):
    wid = lax.axis_index("s") * 2 + lax.axis_index("c")
    base = wid * SIZE
    iota16 = lax.iota(jnp.int32, 16)

    def chunk_body(k, _):
        off = base + k * CH
        pltpu.sync_copy(winner_hbm.at[pl.ds(off, CH)], wbuf)

        def fix_body(j, _):
            w = wbuf[pl.ds(j * 16, 16)]
            good = (w & 1) == 0
            pos = off + j * 16 + iota16
            idx0[pl.ds(j * 16, 16)] = jnp.where(good, w >> 1, pos)
            return ()

        lax.fori_loop(0, CH // 16, fix_body, ())
        pltpu.async_copy(img0_hbm.at[idx0], r0, sem).wait()
        pltpu.async_copy(img1_hbm.at[idx0], r1, sem).wait()
        pltpu.async_copy(img2_hbm.at[idx0], r2, sem).wait()
        zero16 = jnp.zeros((16,), jnp.float32)

        def mix_body(j, _):
            w = wbuf[pl.ds(j * 16, 16)]
            good = (w & 1) == 0
            pos3 = (j * 16 + iota16) * 3
            v0 = jnp.where(good, r0[pl.ds(j * 16, 16)], zero16)
            v1 = jnp.where(good, r1[pl.ds(j * 16, 16)], zero16)
            v2 = jnp.where(good, r2[pl.ds(j * 16, 16)], zero16)
            plsc.store_scatter(rows, [pos3], v0)
            plsc.store_scatter(rows, [pos3 + 1], v1)
            plsc.store_scatter(rows, [pos3 + 2], v2)
            return ()

        lax.fori_loop(0, CH // 16, mix_body, ())
        pltpu.sync_copy(rows, out_hbm.at[pl.ds(off * 3, CH * 3)])
        return ()

    lax.fori_loop(0, SIZE // CH, chunk_body, ())


@jax.jit
def kernel(img, flow, depth, split):
    fx = flow[0, :, :, 0]
    fy = flow[0, :, :, 1]
    maxd = pl.pallas_call(
        _tc_maxd,
        grid=(H // BLK,),
        in_specs=[pl.BlockSpec((BLK, W), lambda i: (i, 0))],
        out_specs=pl.BlockSpec(memory_space=pltpu.SMEM, block_shape=(1, 1),
                               index_map=lambda i: (0, 0)),
        out_shape=jax.ShapeDtypeStruct((1, 1), jnp.float32),
    )(depth)
    tlin = pl.pallas_call(
        _tc_prep,
        grid=(H // BLK,),
        in_specs=[
            pl.BlockSpec(memory_space=pltpu.SMEM, block_shape=(1, 1),
                         index_map=lambda i: (0, 0)),
            pl.BlockSpec((BLK, W), lambda i: (i, 0)),
            pl.BlockSpec((BLK, W), lambda i: (i, 0)),
            pl.BlockSpec((BLK, W), lambda i: (i, 0)),
        ],
        out_specs=pl.BlockSpec((BLK, W), lambda i: (i, 0)),
        out_shape=jax.ShapeDtypeStruct((H, W), jnp.int32),
    )(maxd, fx, fy, depth)

    mesh = plsc.VectorSubcoreMesh(core_axis_name="c", subcore_axis_name="s")

    sc_scatter = pl.kernel(
        _sc_scatter_body,
        out_type=jax.ShapeDtypeStruct((HW,), jnp.int32),
        mesh=mesh,
        scratch_types=[
            pltpu.VMEM((SIZE,), jnp.int32),
            pltpu.VMEM((WIN,), jnp.int32),
        ],
        compiler_params=pltpu.CompilerParams(needs_layout_passes=False),
    )
    winner = sc_scatter(tlin.reshape(HW))

    sc_gather = pl.kernel(
        _sc_gather_body,
        out_type=jax.ShapeDtypeStruct((HW * C,), jnp.float32),
        mesh=mesh,
        scratch_types=[
            pltpu.VMEM((CH,), jnp.int32),
            pltpu.VMEM((CH,), jnp.int32),
            pltpu.VMEM((CH,), jnp.float32),
            pltpu.VMEM((CH,), jnp.float32),
            pltpu.VMEM((CH,), jnp.float32),
            pltpu.VMEM((CH * C,), jnp.float32),
            pltpu.SemaphoreType.DMA,
        ],
        compiler_params=pltpu.CompilerParams(needs_layout_passes=False),
    )
    out = sc_gather(winner,
                    img[:, :, 0].reshape(HW),
                    img[:, :, 1].reshape(HW),
                    img[:, :, 2].reshape(HW))
    return out.reshape(H, W, C)


# speedup vs baseline: 5.4608x; 1.6965x over previous
import jax
import jax.numpy as jnp
from jax import lax
from jax.experimental import pallas as pl
from jax.experimental.pallas import tpu as pltpu
from jax.experimental.pallas import tpu_sc as plsc

H, W, C = 1080, 1920, 3
HW = H * W
BLK = 8
NTILES = 32
SIZE = HW // NTILES
BAND = 64
NRWIN = 168
SUB = 8 * W
NSUB = NRWIN // 8
CH = 6480
TMASK = (1 << 21) - 1
FLAG = 1 << 21


def _tc_maxd(depth_ref, maxd_ref):
    i = pl.program_id(0)
    bm = jnp.max(depth_ref[...])

    @pl.when(i == 0)
    def _():
        maxd_ref[0, 0] = bm

    @pl.when(i > 0)
    def _():
        maxd_ref[0, 0] = jnp.maximum(maxd_ref[0, 0], bm)


def _tc_prep(maxd_ref, fx_ref, fy_ref, depth_ref, tlin_ref, far_ref):
    i = pl.program_id(0)
    fx = fx_ref[...]
    fy = fy_ref[...]
    ysi = lax.broadcasted_iota(jnp.int32, (BLK, W), 0) + i * BLK
    xs = lax.broadcasted_iota(jnp.int32, (BLK, W), 1).astype(jnp.float32)
    ys = ysi.astype(jnp.float32)
    tx = jnp.round(xs + fx).astype(jnp.int32)
    ty = jnp.round(ys + fy).astype(jnp.int32)
    valid = (tx >= 0) & (tx < W) & (ty >= 0) & (ty < H)
    bad = (depth_ref[...] == maxd_ref[0, 0]).astype(jnp.int32)
    tlin_ref[...] = jnp.where(valid, ty * W + tx, HW) | (bad << 21)
    far = jnp.sum((valid & (jnp.abs(ty - ysi) > BAND)).astype(jnp.int32))

    @pl.when(i == 0)
    def _():
        far_ref[0, 0] = far

    @pl.when(i > 0)
    def _():
        far_ref[0, 0] = far_ref[0, 0] + far


def _sc_scatter_body(tlin_hbm, far_hbm, winner_hbm, wloc, buf0, buf1, farv,
                     sem0, sem1):
    wid = lax.axis_index("s") * 2 + lax.axis_index("c")
    base = wid * SIZE
    neg1 = jnp.full((16,), -1, jnp.int32)

    def init_body(j, _):
        wloc[pl.ds(j * 16, 16)] = neg1
        return ()

    lax.fori_loop(0, SIZE // 16, init_body, ())
    iota16 = lax.iota(jnp.int32, 16)

    def make_vbody(buf, sbase):
        def vbody(j, _):
            t = buf[pl.ds(j * 16, 16)]
            local = (t & TMASK) - base
            inr = (local >= 0) & (local < SIZE)

            @pl.when(jnp.any(inr))
            def _():
                _, lastocc = plsc.scan_count(local, inr)
                keep = lastocc & inr
                s = sbase + j * 16 + iota16
                sval = s * 2 + (t >> 21)
                plsc.store_scatter(wloc, [local], sval, mask=keep)

            return ()

        return vbody

    r0 = jnp.clip(((wid * 135) >> 2) - (BAND + 1), 0, H - NRWIN)
    nbase = r0 * W
    bufs = (buf0, buf1)
    sems = (sem0, sem1)
    handles = [None] * NSUB
    handles[0] = pltpu.async_copy(
        tlin_hbm.at[pl.ds(nbase, SUB)], buf0, sem0)
    for u in range(NSUB):
        if u + 1 < NSUB:
            handles[u + 1] = pltpu.async_copy(
                tlin_hbm.at[pl.ds(nbase + (u + 1) * SUB, SUB)],
                bufs[(u + 1) % 2], sems[(u + 1) % 2])
        handles[u].wait()
        lax.fori_loop(0, SUB // 16, make_vbody(bufs[u % 2], nbase + u * SUB),
                      ())

    pltpu.sync_copy(far_hbm, farv)

    @pl.when(jnp.any(farv[...] != 0))
    def _():
        def win_body(w, _):
            pltpu.sync_copy(tlin_hbm.at[pl.ds(w * SUB, SUB)], buf0)
            lax.fori_loop(0, SUB // 16, make_vbody(buf0, w * SUB), ())
            return ()

        lax.fori_loop(0, HW // SUB, win_body, ())

    pltpu.sync_copy(wloc, winner_hbm.at[pl.ds(base, SIZE)])


def _sc_gather_body(winner_hbm, img0_hbm, img1_hbm, img2_hbm, out_hbm,
                    wbuf, idx0, r0, r1, r2, rows, sem):
    wid = lax.axis_index("s") * 2 + lax.axis_index("c")
    base = wid * SIZE
    iota16 = lax.iota(jnp.int32, 16)

    def chunk_body(k, _):
        off = base + k * CH
        pltpu.sync_copy(winner_hbm.at[pl.ds(off, CH)], wbuf)

        def fix_body(j, _):
            w = wbuf[pl.ds(j * 16, 16)]
            good = (w & 1) == 0
            pos = off + j * 16 + iota16
            idx0[pl.ds(j * 16, 16)] = jnp.where(good, w >> 1, pos)
            return ()

        lax.fori_loop(0, CH // 16, fix_body, ())
        pltpu.async_copy(img0_hbm.at[idx0], r0, sem).wait()
        pltpu.async_copy(img1_hbm.at[idx0], r1, sem).wait()
        pltpu.async_copy(img2_hbm.at[idx0], r2, sem).wait()
        zero16 = jnp.zeros((16,), jnp.float32)

        def mix_body(j, _):
            w = wbuf[pl.ds(j * 16, 16)]
            good = (w & 1) == 0
            pos3 = (j * 16 + iota16) * 3
            v0 = jnp.where(good, r0[pl.ds(j * 16, 16)], zero16)
            v1 = jnp.where(good, r1[pl.ds(j * 16, 16)], zero16)
            v2 = jnp.where(good, r2[pl.ds(j * 16, 16)], zero16)
            plsc.store_scatter(rows, [pos3], v0)
            plsc.store_scatter(rows, [pos3 + 1], v1)
            plsc.store_scatter(rows, [pos3 + 2], v2)
            return ()

        lax.fori_loop(0, CH // 16, mix_body, ())
        pltpu.sync_copy(rows, out_hbm.at[pl.ds(off * 3, CH * 3)])
        return ()

    lax.fori_loop(0, SIZE // CH, chunk_body, ())


@jax.jit
def kernel(img, flow, depth, split):
    fx = flow[0, :, :, 0]
    fy = flow[0, :, :, 1]
    maxd = pl.pallas_call(
        _tc_maxd,
        grid=(H // BLK,),
        in_specs=[pl.BlockSpec((BLK, W), lambda i: (i, 0))],
        out_specs=pl.BlockSpec(memory_space=pltpu.SMEM, block_shape=(1, 1),
                               index_map=lambda i: (0, 0)),
        out_shape=jax.ShapeDtypeStruct((1, 1), jnp.float32),
    )(depth)
    tlin, farcnt = pl.pallas_call(
        _tc_prep,
        grid=(H // BLK,),
        in_specs=[
            pl.BlockSpec(memory_space=pltpu.SMEM, block_shape=(1, 1),
                         index_map=lambda i: (0, 0)),
            pl.BlockSpec((BLK, W), lambda i: (i, 0)),
            pl.BlockSpec((BLK, W), lambda i: (i, 0)),
            pl.BlockSpec((BLK, W), lambda i: (i, 0)),
        ],
        out_specs=[
            pl.BlockSpec((BLK, W), lambda i: (i, 0)),
            pl.BlockSpec(memory_space=pltpu.SMEM, block_shape=(1, 1),
                         index_map=lambda i: (0, 0)),
        ],
        out_shape=[
            jax.ShapeDtypeStruct((H, W), jnp.int32),
            jax.ShapeDtypeStruct((1, 1), jnp.int32),
        ],
    )(maxd, fx, fy, depth)
    far16 = jnp.broadcast_to(farcnt.reshape(()), (16,))

    mesh = plsc.VectorSubcoreMesh(core_axis_name="c", subcore_axis_name="s")

    sc_scatter = pl.kernel(
        _sc_scatter_body,
        out_type=jax.ShapeDtypeStruct((HW,), jnp.int32),
        mesh=mesh,
        scratch_types=[
            pltpu.VMEM((SIZE,), jnp.int32),
            pltpu.VMEM((SUB,), jnp.int32),
            pltpu.VMEM((SUB,), jnp.int32),
            pltpu.VMEM((16,), jnp.int32),
            pltpu.SemaphoreType.DMA,
            pltpu.SemaphoreType.DMA,
        ],
        compiler_params=pltpu.CompilerParams(needs_layout_passes=False),
    )
    winner = sc_scatter(tlin.reshape(HW), far16)

    sc_gather = pl.kernel(
        _sc_gather_body,
        out_type=jax.ShapeDtypeStruct((HW * C,), jnp.float32),
        mesh=mesh,
        scratch_types=[
            pltpu.VMEM((CH,), jnp.int32),
            pltpu.VMEM((CH,), jnp.int32),
            pltpu.VMEM((CH,), jnp.float32),
            pltpu.VMEM((CH,), jnp.float32),
            pltpu.VMEM((CH,), jnp.float32),
            pltpu.VMEM((CH * C,), jnp.float32),
            pltpu.SemaphoreType.DMA,
        ],
        compiler_params=pltpu.CompilerParams(needs_layout_passes=False),
    )
    out = sc_gather(winner,
                    img[:, :, 0].reshape(HW),
                    img[:, :, 1].reshape(HW),
                    img[:, :, 2].reshape(HW))
    return out.reshape(H, W, C)


# speedup vs baseline: 5.5446x; 1.0153x over previous
import jax
import jax.numpy as jnp
from jax import lax
from jax.experimental import pallas as pl
from jax.experimental.pallas import tpu as pltpu
from jax.experimental.pallas import tpu_sc as plsc

H, W, C = 1080, 1920, 3
HW = H * W
BLK = 8
NTILES = 32
SIZE = HW // NTILES
BAND = 64
NRWIN = 168
SUB = 4 * W
NSUB = NRWIN // 4
CH = 3240
TMASK = (1 << 21) - 1
FLAG = 1 << 21


def _tc_maxd(depth_ref, maxd_ref):
    i = pl.program_id(0)
    bm = jnp.max(depth_ref[...])

    @pl.when(i == 0)
    def _():
        maxd_ref[0, 0] = bm

    @pl.when(i > 0)
    def _():
        maxd_ref[0, 0] = jnp.maximum(maxd_ref[0, 0], bm)


def _tc_prep(maxd_ref, fx_ref, fy_ref, depth_ref, tlin_ref, far_ref):
    i = pl.program_id(0)
    fx = fx_ref[...]
    fy = fy_ref[...]
    ysi = lax.broadcasted_iota(jnp.int32, (BLK, W), 0) + i * BLK
    xs = lax.broadcasted_iota(jnp.int32, (BLK, W), 1).astype(jnp.float32)
    ys = ysi.astype(jnp.float32)
    tx = jnp.round(xs + fx).astype(jnp.int32)
    ty = jnp.round(ys + fy).astype(jnp.int32)
    valid = (tx >= 0) & (tx < W) & (ty >= 0) & (ty < H)
    bad = (depth_ref[...] == maxd_ref[0, 0]).astype(jnp.int32)
    tlin_ref[...] = jnp.where(valid, ty * W + tx, HW) | (bad << 21)
    far = jnp.sum((valid & (jnp.abs(ty - ysi) > BAND)).astype(jnp.int32))

    @pl.when(i == 0)
    def _():
        far_ref[0, 0] = far

    @pl.when(i > 0)
    def _():
        far_ref[0, 0] = far_ref[0, 0] + far


def _sc_main_body(tlin_hbm, far_hbm, img0_hbm, img1_hbm, img2_hbm, out_hbm,
                  wloc, buf0, buf1, farv, idxA, idxB, r0A, r1A, r2A,
                  r0B, r1B, r2B, rows, sem0, sem1, gsemA, gsemB):
    wid = lax.axis_index("s") * 2 + lax.axis_index("c")
    base = wid * SIZE
    neg1 = jnp.full((16,), -1, jnp.int32)

    def init_body(j, _):
        wloc[pl.ds(j * 16, 16)] = neg1
        return ()

    lax.fori_loop(0, SIZE // 16, init_body, ())
    iota16 = lax.iota(jnp.int32, 16)

    def make_vbody(buf, sbase):
        def vbody(j, _):
            t = buf[pl.ds(j * 16, 16)]
            local = (t & TMASK) - base
            inr = (local >= 0) & (local < SIZE)

            @pl.when(jnp.any(inr))
            def _():
                _, lastocc = plsc.scan_count(local, inr)
                keep = lastocc & inr
                s = sbase + j * 16 + iota16
                sval = s * 2 + (t >> 21)
                plsc.store_scatter(wloc, [local], sval, mask=keep)

            return ()

        return vbody

    r0 = jnp.clip(((wid * 135) >> 2) - (BAND + 1), 0, H - NRWIN)
    nbase = r0 * W
    bufs = (buf0, buf1)
    sems = (sem0, sem1)
    handles = [None] * NSUB
    handles[0] = pltpu.async_copy(
        tlin_hbm.at[pl.ds(nbase, SUB)], buf0, sem0)
    for u in range(NSUB):
        if u + 1 < NSUB:
            handles[u + 1] = pltpu.async_copy(
                tlin_hbm.at[pl.ds(nbase + (u + 1) * SUB, SUB)],
                bufs[(u + 1) % 2], sems[(u + 1) % 2])
        handles[u].wait()
        lax.fori_loop(0, SUB // 16, make_vbody(bufs[u % 2], nbase + u * SUB),
                      ())

    pltpu.sync_copy(far_hbm, farv)

    @pl.when(jnp.any(farv[...] != 0))
    def _():
        def win_body(w, _):
            pltpu.sync_copy(tlin_hbm.at[pl.ds(w * SUB, SUB)], buf0)
            lax.fori_loop(0, SUB // 16, make_vbody(buf0, w * SUB), ())
            return ()

        lax.fori_loop(0, HW // SUB, win_body, ())

    iota = iota16

    def make_fix(k, idx):
        def fix_body(j, _):
            w = wloc[pl.ds(k * CH + j * 16, 16)]
            good = (w & 1) == 0
            pos = base + k * CH + j * 16 + iota
            idx[pl.ds(j * 16, 16)] = jnp.where(good, w >> 1, pos)
            return ()

        return fix_body

    def fire(idx, r0, r1, r2, sem):
        h0 = pltpu.async_copy(img0_hbm.at[idx], r0, sem)
        h1 = pltpu.async_copy(img1_hbm.at[idx], r1, sem)
        h2 = pltpu.async_copy(img2_hbm.at[idx], r2, sem)
        return (h0, h1, h2)

    zero16 = jnp.zeros((16,), jnp.float32)

    def make_mix(k, r0, r1, r2):
        def mix_body(j, _):
            w = wloc[pl.ds(k * CH + j * 16, 16)]
            good = (w & 1) == 0
            pos3 = (j * 16 + iota) * 3
            v0 = jnp.where(good, r0[pl.ds(j * 16, 16)], zero16)
            v1 = jnp.where(good, r1[pl.ds(j * 16, 16)], zero16)
            v2 = jnp.where(good, r2[pl.ds(j * 16, 16)], zero16)
            plsc.store_scatter(rows, [pos3], v0)
            plsc.store_scatter(rows, [pos3 + 1], v1)
            plsc.store_scatter(rows, [pos3 + 2], v2)
            return ()

        return mix_body

    def mix_out(k, r0, r1, r2):
        lax.fori_loop(0, CH // 16, make_mix(k, r0, r1, r2), ())
        pltpu.sync_copy(rows, out_hbm.at[pl.ds((base + k * CH) * 3, CH * 3)])

    bufsets = ((idxA, r0A, r1A, r2A, gsemA), (idxB, r0B, r1B, r2B, gsemB))
    prev = None
    for k in range(SIZE // CH):
        idx, r0, r1, r2, gsem = bufsets[k % 2]
        lax.fori_loop(0, CH // 16, make_fix(k, idx), ())
        handles = fire(idx, r0, r1, r2, gsem)
        if prev is not None:
            for h in prev[1]:
                h.wait()
            pidx, pr0, pr1, pr2, _ = bufsets[(k - 1) % 2]
            mix_out(prev[0], pr0, pr1, pr2)
        prev = (k, handles)
    for h in prev[1]:
        h.wait()
    _, pr0, pr1, pr2, _ = bufsets[prev[0] % 2]
    mix_out(prev[0], pr0, pr1, pr2)


@jax.jit
def kernel(img, flow, depth, split):
    fx = flow[0, :, :, 0]
    fy = flow[0, :, :, 1]
    maxd = pl.pallas_call(
        _tc_maxd,
        grid=(H // BLK,),
        in_specs=[pl.BlockSpec((BLK, W), lambda i: (i, 0))],
        out_specs=pl.BlockSpec(memory_space=pltpu.SMEM, block_shape=(1, 1),
                               index_map=lambda i: (0, 0)),
        out_shape=jax.ShapeDtypeStruct((1, 1), jnp.float32),
    )(depth)
    tlin, farcnt = pl.pallas_call(
        _tc_prep,
        grid=(H // BLK,),
        in_specs=[
            pl.BlockSpec(memory_space=pltpu.SMEM, block_shape=(1, 1),
                         index_map=lambda i: (0, 0)),
            pl.BlockSpec((BLK, W), lambda i: (i, 0)),
            pl.BlockSpec((BLK, W), lambda i: (i, 0)),
            pl.BlockSpec((BLK, W), lambda i: (i, 0)),
        ],
        out_specs=[
            pl.BlockSpec((BLK, W), lambda i: (i, 0)),
            pl.BlockSpec(memory_space=pltpu.SMEM, block_shape=(1, 1),
                         index_map=lambda i: (0, 0)),
        ],
        out_shape=[
            jax.ShapeDtypeStruct((H, W), jnp.int32),
            jax.ShapeDtypeStruct((1, 1), jnp.int32),
        ],
    )(maxd, fx, fy, depth)
    far16 = jnp.broadcast_to(farcnt.reshape(()), (16,))

    mesh = plsc.VectorSubcoreMesh(core_axis_name="c", subcore_axis_name="s")

    sc_main = pl.kernel(
        _sc_main_body,
        out_type=jax.ShapeDtypeStruct((HW * C,), jnp.float32),
        mesh=mesh,
        scratch_types=[
            pltpu.VMEM((SIZE,), jnp.int32),
            pltpu.VMEM((SUB,), jnp.int32),
            pltpu.VMEM((SUB,), jnp.int32),
            pltpu.VMEM((16,), jnp.int32),
            pltpu.VMEM((CH,), jnp.int32),
            pltpu.VMEM((CH,), jnp.int32),
            pltpu.VMEM((CH,), jnp.float32),
            pltpu.VMEM((CH,), jnp.float32),
            pltpu.VMEM((CH,), jnp.float32),
            pltpu.VMEM((CH,), jnp.float32),
            pltpu.VMEM((CH,), jnp.float32),
            pltpu.VMEM((CH,), jnp.float32),
            pltpu.VMEM((CH * C,), jnp.float32),
            pltpu.SemaphoreType.DMA,
            pltpu.SemaphoreType.DMA,
            pltpu.SemaphoreType.DMA,
            pltpu.SemaphoreType.DMA,
        ],
        compiler_params=pltpu.CompilerParams(needs_layout_passes=False),
    )
    out = sc_main(tlin.reshape(HW), far16,
                  img[:, :, 0].reshape(HW),
                  img[:, :, 1].reshape(HW),
                  img[:, :, 2].reshape(HW))
    return out.reshape(H, W, C)


# speedup vs baseline: 5.5513x; 1.0012x over previous
import jax
import jax.numpy as jnp
from jax import lax
from jax.experimental import pallas as pl
from jax.experimental.pallas import tpu as pltpu
from jax.experimental.pallas import tpu_sc as plsc

H, W, C = 1080, 1920, 3
HW = H * W
BLK = 8
NTILES = 32
SIZE = HW // NTILES
BAND = 64
NRWIN = 168
SUB = 4 * W
NSUB = NRWIN // 4
CH = 3600
TMASK = (1 << 21) - 1
FLAG = 1 << 21


def _tc_maxd(depth_ref, maxd_ref):
    i = pl.program_id(0)
    bm = jnp.max(depth_ref[...])

    @pl.when(i == 0)
    def _():
        maxd_ref[0, 0] = bm

    @pl.when(i > 0)
    def _():
        maxd_ref[0, 0] = jnp.maximum(maxd_ref[0, 0], bm)


def _tc_prep(maxd_ref, fx_ref, fy_ref, depth_ref, tlin_ref, far_ref):
    i = pl.program_id(0)
    fx = fx_ref[...]
    fy = fy_ref[...]
    ysi = lax.broadcasted_iota(jnp.int32, (BLK, W), 0) + i * BLK
    xs = lax.broadcasted_iota(jnp.int32, (BLK, W), 1).astype(jnp.float32)
    ys = ysi.astype(jnp.float32)
    tx = jnp.round(xs + fx).astype(jnp.int32)
    ty = jnp.round(ys + fy).astype(jnp.int32)
    valid = (tx >= 0) & (tx < W) & (ty >= 0) & (ty < H)
    bad = (depth_ref[...] == maxd_ref[0, 0]).astype(jnp.int32)
    tlin_ref[...] = jnp.where(valid, ty * W + tx, HW) | (bad << 21)
    far = jnp.sum((valid & (jnp.abs(ty - ysi) > BAND)).astype(jnp.int32))

    @pl.when(i == 0)
    def _():
        far_ref[0, 0] = far

    @pl.when(i > 0)
    def _():
        far_ref[0, 0] = far_ref[0, 0] + far


def _sc_main_body(tlin_hbm, far_hbm, img0_hbm, img1_hbm, img2_hbm, out_hbm,
                  wloc, buf0, buf1, farv, idxA, idxB, r0A, r1A, r2A,
                  r0B, r1B, r2B, rows, sem0, sem1, gsemA, gsemB):
    wid = lax.axis_index("s") * 2 + lax.axis_index("c")
    base = wid * SIZE
    neg1 = jnp.full((16,), -1, jnp.int32)

    def init_body(j, _):
        wloc[pl.ds(j * 16, 16)] = neg1
        return ()

    lax.fori_loop(0, SIZE // 16, init_body, ())
    iota16 = lax.iota(jnp.int32, 16)

    def make_vbody(buf, sbase):
        def vbody(j, _):
            t = buf[pl.ds(j * 16, 16)]
            local = (t & TMASK) - base
            inr = (local >= 0) & (local < SIZE)

            @pl.when(jnp.any(inr))
            def _():
                _, lastocc = plsc.scan_count(local, inr)
                keep = lastocc & inr
                s = sbase + j * 16 + iota16
                sval = s * 2 + (t >> 21)
                plsc.store_scatter(wloc, [local], sval, mask=keep)

            return ()

        return vbody

    r0 = jnp.clip(((wid * 135) >> 2) - (BAND + 1), 0, H - NRWIN)
    nbase = r0 * W
    bufs = (buf0, buf1)
    sems = (sem0, sem1)
    handles = [None] * NSUB
    handles[0] = pltpu.async_copy(
        tlin_hbm.at[pl.ds(nbase, SUB)], buf0, sem0)
    for u in range(NSUB):
        if u + 1 < NSUB:
            handles[u + 1] = pltpu.async_copy(
                tlin_hbm.at[pl.ds(nbase + (u + 1) * SUB, SUB)],
                bufs[(u + 1) % 2], sems[(u + 1) % 2])
        handles[u].wait()
        lax.fori_loop(0, SUB // 16, make_vbody(bufs[u % 2], nbase + u * SUB),
                      ())

    pltpu.sync_copy(far_hbm, farv)

    @pl.when(jnp.any(farv[...] != 0))
    def _():
        def win_body(w, _):
            pltpu.sync_copy(tlin_hbm.at[pl.ds(w * SUB, SUB)], buf0)
            lax.fori_loop(0, SUB // 16, make_vbody(buf0, w * SUB), ())
            return ()

        lax.fori_loop(0, HW // SUB, win_body, ())

    iota = iota16

    def make_fix(k, idx):
        def fix_body(j, _):
            w = wloc[pl.ds(k * CH + j * 16, 16)]
            good = (w & 1) == 0
            pos = base + k * CH + j * 16 + iota
            idx[pl.ds(j * 16, 16)] = jnp.where(good, w >> 1, pos)
            return ()

        return fix_body

    def fire(idx, r0, r1, r2, sem):
        h0 = pltpu.async_copy(img0_hbm.at[idx], r0, sem)
        h1 = pltpu.async_copy(img1_hbm.at[idx], r1, sem)
        h2 = pltpu.async_copy(img2_hbm.at[idx], r2, sem)
        return (h0, h1, h2)

    zero16 = jnp.zeros((16,), jnp.float32)

    def make_mix(k, r0, r1, r2):
        def mix_body(j, _):
            w = wloc[pl.ds(k * CH + j * 16, 16)]
            good = (w & 1) == 0
            pos3 = (j * 16 + iota) * 3
            v0 = jnp.where(good, r0[pl.ds(j * 16, 16)], zero16)
            v1 = jnp.where(good, r1[pl.ds(j * 16, 16)], zero16)
            v2 = jnp.where(good, r2[pl.ds(j * 16, 16)], zero16)
            plsc.store_scatter(rows, [pos3], v0)
            plsc.store_scatter(rows, [pos3 + 1], v1)
            plsc.store_scatter(rows, [pos3 + 2], v2)
            return ()

        return mix_body

    def mix_out(k, r0, r1, r2):
        lax.fori_loop(0, CH // 16, make_mix(k, r0, r1, r2), ())
        pltpu.sync_copy(rows, out_hbm.at[pl.ds((base + k * CH) * 3, CH * 3)])

    bufsets = ((idxA, r0A, r1A, r2A, gsemA), (idxB, r0B, r1B, r2B, gsemB))
    prev = None
    for k in range(SIZE // CH):
        idx, r0, r1, r2, gsem = bufsets[k % 2]
        lax.fori_loop(0, CH // 16, make_fix(k, idx), ())
        handles = fire(idx, r0, r1, r2, gsem)
        if prev is not None:
            for h in prev[1]:
                h.wait()
            _, pr0, pr1, pr2, _ = bufsets[(k - 1) % 2]
            mix_out(prev[0], pr0, pr1, pr2)
        prev = (k, handles)
    for h in prev[1]:
        h.wait()
    _, pr0, pr1, pr2, _ = bufsets[prev[0] % 2]
    mix_out(prev[0], pr0, pr1, pr2)


@jax.jit
def kernel(img, flow, depth, split):
    fx = flow[0, :, :, 0]
    fy = flow[0, :, :, 1]
    maxd = pl.pallas_call(
        _tc_maxd,
        grid=(H // BLK,),
        in_specs=[pl.BlockSpec((BLK, W), lambda i: (i, 0))],
        out_specs=pl.BlockSpec(memory_space=pltpu.SMEM, block_shape=(1, 1),
                               index_map=lambda i: (0, 0)),
        out_shape=jax.ShapeDtypeStruct((1, 1), jnp.float32),
    )(depth)
    tlin, farcnt = pl.pallas_call(
        _tc_prep,
        grid=(H // BLK,),
        in_specs=[
            pl.BlockSpec(memory_space=pltpu.SMEM, block_shape=(1, 1),
                         index_map=lambda i: (0, 0)),
            pl.BlockSpec((BLK, W), lambda i: (i, 0)),
            pl.BlockSpec((BLK, W), lambda i: (i, 0)),
            pl.BlockSpec((BLK, W), lambda i: (i, 0)),
        ],
        out_specs=[
            pl.BlockSpec((BLK, W), lambda i: (i, 0)),
            pl.BlockSpec(memory_space=pltpu.SMEM, block_shape=(1, 1),
                         index_map=lambda i: (0, 0)),
        ],
        out_shape=[
            jax.ShapeDtypeStruct((H, W), jnp.int32),
            jax.ShapeDtypeStruct((1, 1), jnp.int32),
        ],
    )(maxd, fx, fy, depth)
    far16 = jnp.broadcast_to(farcnt.reshape(()), (16,))

    mesh = plsc.VectorSubcoreMesh(core_axis_name="c", subcore_axis_name="s")

    sc_main = pl.kernel(
        _sc_main_body,
        out_type=jax.ShapeDtypeStruct((HW * C,), jnp.float32),
        mesh=mesh,
        scratch_types=[
            pltpu.VMEM((SIZE,), jnp.int32),
            pltpu.VMEM((SUB,), jnp.int32),
            pltpu.VMEM((SUB,), jnp.int32),
            pltpu.VMEM((16,), jnp.int32),
            pltpu.VMEM((CH,), jnp.int32),
            pltpu.VMEM((CH,), jnp.int32),
            pltpu.VMEM((CH,), jnp.float32),
            pltpu.VMEM((CH,), jnp.float32),
            pltpu.VMEM((CH,), jnp.float32),
            pltpu.VMEM((CH,), jnp.float32),
            pltpu.VMEM((CH,), jnp.float32),
            pltpu.VMEM((CH,), jnp.float32),
            pltpu.VMEM((CH * C,), jnp.float32),
            pltpu.SemaphoreType.DMA,
            pltpu.SemaphoreType.DMA,
            pltpu.SemaphoreType.DMA,
            pltpu.SemaphoreType.DMA,
        ],
        compiler_params=pltpu.CompilerParams(needs_layout_passes=False),
    )
    out = sc_main(tlin.reshape(HW), far16,
                  img[:, :, 0].reshape(HW),
                  img[:, :, 1].reshape(HW),
                  img[:, :, 2].reshape(HW))
    return out.reshape(H, W, C)
